# Initial kernel scaffold; baseline (speedup 1.0000x reference)
import jax
import jax.numpy as jnp
from jax.experimental import pallas as pl


def kernel(t, embeddings, W1, b1, W2, b2):
    raise NotImplementedError("write your pallas kernel here")


# speedup vs baseline: 3.2606x; 3.2606x over previous
import functools

import jax
import jax.numpy as jnp
from jax import lax
from jax.experimental import pallas as pl
from jax.experimental.pallas import tpu as pltpu
from jax.experimental.pallas import tpu_sc as plsc

_VOCAB = 100000
_EMBED = 64
_N_FEAT = 36
_HIDDEN = 1024
_N_CLASSES = 3
_BATCH = 4096

_NC = 2
_NS = 16
_NW = _NC * _NS
_ROWS = _BATCH * _N_FEAT
_CHUNK = 128
_N_CHUNKS = _ROWS // _CHUNK
_CHUNKS_PER_W = _N_CHUNKS // _NW


def _sc_gather_body(idx_hbm, table_hbm, out_hbm, idx_v, rows_v, sem):
    wid = lax.axis_index("s") * _NC + lax.axis_index("c")
    base = wid * _CHUNKS_PER_W
    pltpu.sync_copy(idx_hbm.at[wid], idx_v)

    def body(j, carry):
        pltpu.async_copy(table_hbm.at[idx_v.at[j]], rows_v, sem).wait()
        pltpu.sync_copy(rows_v, out_hbm.at[pl.ds((base + j) * _CHUNK, _CHUNK)])
        return carry

    lax.fori_loop(0, _CHUNKS_PER_W, body, 0)


_sc_gather = functools.partial(
    pl.kernel,
    mesh=plsc.VectorSubcoreMesh(core_axis_name="c", subcore_axis_name="s"),
    out_type=jax.ShapeDtypeStruct((_ROWS, _EMBED), jnp.float32),
    scratch_types=[
        pltpu.VMEM((_CHUNKS_PER_W, _CHUNK), jnp.int32),
        pltpu.VMEM((_CHUNK, _EMBED), jnp.float32),
        pltpu.SemaphoreType.DMA,
    ],
    compiler_params=pltpu.CompilerParams(use_tc_tiling_on_sc=False),
)(_sc_gather_body)


_BB = 256


def _tc_mlp_body(x_ref, w1t_ref, b1_ref, w2t_ref, b2_ref, out_ref):
    h = jnp.dot(x_ref[...], w1t_ref[...], preferred_element_type=jnp.float32)
    h = jnp.maximum(h + b1_ref[...], 0.0)
    out = jnp.dot(h, w2t_ref[...], preferred_element_type=jnp.float32)
    out_ref[...] = out + b2_ref[...]


def _tc_mlp(x, w1t, b1, w2t_pad, b2_pad):
    return pl.pallas_call(
        _tc_mlp_body,
        grid=(_BATCH // _BB,),
        in_specs=[
            pl.BlockSpec((_BB, _N_FEAT * _EMBED), lambda i: (i, 0)),
            pl.BlockSpec((_N_FEAT * _EMBED, _HIDDEN), lambda i: (0, 0)),
            pl.BlockSpec((1, _HIDDEN), lambda i: (0, 0)),
            pl.BlockSpec((_HIDDEN, 128), lambda i: (0, 0)),
            pl.BlockSpec((1, 128), lambda i: (0, 0)),
        ],
        out_specs=pl.BlockSpec((_BB, 128), lambda i: (i, 0)),
        out_shape=jax.ShapeDtypeStruct((_BATCH, 128), jnp.float32),
    )(x, w1t, b1, w2t_pad, b2_pad)


def kernel(t, embeddings, W1, b1, W2, b2):
    idx = t.astype(jnp.int32).reshape(_NW, _CHUNKS_PER_W, _CHUNK)
    rows = _sc_gather(idx, embeddings)
    x = rows.reshape(_BATCH, _N_FEAT * _EMBED)

    w1t = W1.T
    w2t_pad = jnp.zeros((_HIDDEN, 128), jnp.float32).at[:, :_N_CLASSES].set(W2.T)
    b2_pad = jnp.zeros((128,), jnp.float32).at[:_N_CLASSES].set(b2)
    logits = _tc_mlp(x, w1t, b1.reshape(1, _HIDDEN), w2t_pad,
                     b2_pad.reshape(1, 128))
    return logits[:, :_N_CLASSES]


# speedup vs baseline: 3.9106x; 1.1993x over previous
import functools

import jax
import jax.numpy as jnp
from jax import lax
from jax.experimental import pallas as pl
from jax.experimental.pallas import tpu as pltpu
from jax.experimental.pallas import tpu_sc as plsc

_VOCAB = 100000
_EMBED = 64
_N_FEAT = 36
_HIDDEN = 1024
_N_CLASSES = 3
_BATCH = 4096

_NC = 2
_NS = 16
_NW = _NC * _NS
_NP = _N_FEAT // 2
_BSTRIPE = _BATCH // _NW
_FPAD = 40


def _sc_gather_body(idx_hbm, table_hbm, out_hbm, idx_v, buf, sem):
    wid = lax.axis_index("s") * _NC + lax.axis_index("c")
    b0 = wid * _BSTRIPE
    pltpu.sync_copy(idx_hbm.at[wid], idx_v)

    def body(p, carry):
        bufl, bufr = buf
        cpl = pltpu.async_copy(table_hbm.at[idx_v.at[2 * p]], bufl, sem)
        cpr = pltpu.async_copy(table_hbm.at[idx_v.at[2 * p + 1]], bufr, sem)
        cpl.wait()
        cpr.wait()
        dst = out_hbm.at[pl.ds(p * _BATCH + b0, _BSTRIPE)]
        pltpu.sync_copy(bufl, dst.at[:, 0:_EMBED])
        pltpu.sync_copy(bufr, dst.at[:, _EMBED:128])
        return carry

    lax.fori_loop(0, _NP, body, 0)


_sc_gather = functools.partial(
    pl.kernel,
    mesh=plsc.VectorSubcoreMesh(core_axis_name="c", subcore_axis_name="s"),
    out_type=jax.ShapeDtypeStruct((_NP * _BATCH, 128), jnp.float32),
    scratch_types=[
        pltpu.VMEM((_FPAD, _BSTRIPE), jnp.int32),
        (pltpu.VMEM((_BSTRIPE, _EMBED), jnp.float32),
         pltpu.VMEM((_BSTRIPE, _EMBED), jnp.float32)),
        pltpu.SemaphoreType.DMA,
    ],
    compiler_params=pltpu.CompilerParams(use_tc_tiling_on_sc=False),
)(_sc_gather_body)


_BB = 256
_NT_DIMS = (((1,), (1,)), ((), ()))


def _tc_mlp_body(x_ref, w1_ref, b1_ref, w2_ref, b2_ref, out_ref):
    acc = jnp.broadcast_to(b1_ref[...], (_BB, _HIDDEN))
    for p in range(_NP):
        acc += lax.dot_general(
            x_ref[p], w1_ref[:, 128 * p:128 * (p + 1)], _NT_DIMS,
            preferred_element_type=jnp.float32)
    h = jnp.maximum(acc, 0.0)
    out = lax.dot_general(h, w2_ref[...], _NT_DIMS,
                          preferred_element_type=jnp.float32)
    out_ref[...] = out + b2_ref[...]


def _tc_mlp(x4, w1, b1, w2_pad, b2_pad):
    return pl.pallas_call(
        _tc_mlp_body,
        grid=(_BATCH // _BB,),
        in_specs=[
            pl.BlockSpec((_NP, _BB, 128), lambda i: (0, i, 0)),
            pl.BlockSpec((_HIDDEN, _N_FEAT * _EMBED), lambda i: (0, 0)),
            pl.BlockSpec((1, _HIDDEN), lambda i: (0, 0)),
            pl.BlockSpec((128, _HIDDEN), lambda i: (0, 0)),
            pl.BlockSpec((1, 128), lambda i: (0, 0)),
        ],
        out_specs=pl.BlockSpec((_BB, 128), lambda i: (i, 0)),
        out_shape=jax.ShapeDtypeStruct((_BATCH, 128), jnp.float32),
    )(x4, w1, b1, w2_pad, b2_pad)


def kernel(t, embeddings, W1, b1, W2, b2):
    t32 = t.astype(jnp.int32)
    idx = jnp.zeros((_NW, _FPAD, _BSTRIPE), jnp.int32).at[:, :_N_FEAT, :].set(
        t32.reshape(_NW, _BSTRIPE, _N_FEAT).transpose(0, 2, 1))
    rows = _sc_gather(idx, embeddings)
    x4 = rows.reshape(_NP, _BATCH, 128)

    w2_pad = jnp.zeros((128, _HIDDEN), jnp.float32).at[:_N_CLASSES, :].set(W2)
    b2_pad = jnp.zeros((128,), jnp.float32).at[:_N_CLASSES].set(b2)
    logits = _tc_mlp(x4, W1, b1.reshape(1, _HIDDEN), w2_pad,
                     b2_pad.reshape(1, 128))
    return logits[:, :_N_CLASSES]
